# baseline (speedup 1.0000x reference)
import functools

import jax
import jax.numpy as jnp
from jax import lax
from jax.experimental import pallas as pl
from jax.experimental.pallas import tpu as pltpu
from jax.experimental.pallas import tpu_sc as plsc

N_TOK = 1048576
N_GRAPH = 1024
N_VAL = 64
N_FEAT = 192
EMB_DIM = 64
HID_DIM = 256
N_CLASS = 10

N_CORES = 2
N_SUBCORES = 16
GH = N_GRAPH // N_CORES
HIST_ROWS = GH + 16
TOK_PER_TILE = N_TOK // N_SUBCORES
CHUNK = 2048
N_CHUNK = TOK_PER_TILE // CHUNK
GROUPS = CHUNK // 16
UNROLL = 4


def _sc_hist_body(s_hbm, c_hbm, p_hbm, batch_hbm, zeros_hbm, out_hbm,
                  s_v, c_v, p_v, g_v, hist_v, shared, sem_z, sem_a, sem_b):
    cid = lax.axis_index("c")
    sid = lax.axis_index("s")
    t0 = sid * TOK_PER_TILE
    gbase = cid * GH

    rows_per_tile = GH // N_SUBCORES
    srow = pl.multiple_of(sid * rows_per_tile, 16)
    zdescs = []
    for r in range(HIST_ROWS // 16):
        zdescs.append(pltpu.async_copy(
            zeros_hbm, hist_v.at[pl.ds(r * 16, 16)], sem_z))
    zdescs.append(pltpu.async_copy(zeros_hbm, shared.at[pl.ds(srow, 16)], sem_z))
    zdescs.append(pltpu.async_copy(zeros_hbm, shared.at[pl.ds(srow + 16, 16)], sem_z))
    for d in zdescs:
        d.wait()
    plsc.subcore_barrier()

    viota = lax.iota(jnp.int32, 16)
    ones = jnp.full((16,), 1.0, jnp.float32)
    sems = (sem_a, sem_b)

    def issue(k):
        slot = k % 2
        off = pl.multiple_of(t0 + k * CHUNK, CHUNK)
        sem = sems[slot]
        return [
            pltpu.async_copy(s_hbm.at[pl.ds(off, CHUNK)], s_v.at[slot], sem),
            pltpu.async_copy(c_hbm.at[pl.ds(off, CHUNK)], c_v.at[slot], sem),
            pltpu.async_copy(p_hbm.at[pl.ds(off, CHUNK)], p_v.at[slot], sem),
            pltpu.async_copy(batch_hbm.at[pl.ds(off, CHUNK)], g_v.at[slot], sem),
        ]

    descs = [None, None]
    descs[0] = issue(0)
    gfirst = jnp.int32(0)
    glast = jnp.int32(0)
    for k in range(N_CHUNK):
        slot = k % 2
        if k + 1 < N_CHUNK:
            descs[(k + 1) % 2] = issue(k + 1)
        for d in descs[slot]:
            d.wait()
        gs, cs, ps, gg = s_v.at[slot], c_v.at[slot], p_v.at[slot], g_v.at[slot]

        def grp_body(i, c, gs=gs, cs=cs, ps=ps, gg=gg):
            for u in range(UNROLL):
                j = pl.multiple_of(i * (16 * UNROLL) + u * 16, 16)
                vg = gg[pl.ds(j, 16)]
                gl = vg - gbase
                msk = (gl >= 0) & (gl < GH)
                glc = jnp.minimum(jnp.maximum(gl, 0), GH - 1)
                sval = gs[pl.ds(j, 16)]
                cval = cs[pl.ds(j, 16)]
                pval = ps[pl.ds(j, 16)]
                plsc.addupdate_scatter(hist_v, [glc, sval], ones, mask=msk)
                plsc.addupdate_scatter(hist_v, [glc, cval + N_VAL], ones, mask=msk)
                plsc.addupdate_scatter(hist_v, [glc, pval + 2 * N_VAL], ones, mask=msk)
            return c
        lax.fori_loop(0, GROUPS // UNROLL, grp_body, 0)

        if k == 0:
            gfirst = jnp.min(gg[pl.ds(0, 16)])
        if k == N_CHUNK - 1:
            glast = jnp.max(gg[pl.ds(CHUNK - 16, 16)])

    lo = (jnp.clip(gfirst - gbase, 0, GH) // 16) * 16
    hi = jnp.clip(glast - gbase + 1, 0, GH)
    nwin = (hi - lo + 15) // 16

    def flush_body(t, c):
        r = pl.multiple_of(lo + t * 16, 16)
        rows = jnp.minimum(r + viota, GH - 1)
        pltpu.sync_copy(hist_v.at[pl.ds(r, 16)], shared.at[rows], add=True)
        return c
    lax.fori_loop(0, nwin, flush_body, 0)
    plsc.subcore_barrier()

    def out_body(t, c):
        r = pl.multiple_of(sid * rows_per_tile + t * 16, 16)
        pltpu.sync_copy(shared.at[pl.ds(r, 16)],
                        out_hbm.at[pl.ds(pl.multiple_of(gbase + r, 16), 16)])
        return c
    lax.fori_loop(0, rows_per_tile // 16, out_body, 0)


_sc_hist = functools.partial(
    pl.kernel,
    out_type=jax.ShapeDtypeStruct((N_GRAPH, N_FEAT), jnp.float32),
    mesh=plsc.VectorSubcoreMesh(
        core_axis_name="c", subcore_axis_name="s",
        num_cores=N_CORES, num_subcores=N_SUBCORES,
    ),
    scratch_types=[
        pltpu.VMEM((2, CHUNK), jnp.int32),
        pltpu.VMEM((2, CHUNK), jnp.int32),
        pltpu.VMEM((2, CHUNK), jnp.int32),
        pltpu.VMEM((2, CHUNK), jnp.int32),
        pltpu.VMEM((HIST_ROWS, N_FEAT), jnp.float32),
        pltpu.VMEM_SHARED((GH, N_FEAT), jnp.float32),
        pltpu.SemaphoreType.DMA,
        pltpu.SemaphoreType.DMA,
        pltpu.SemaphoreType.DMA,
    ],
    compiler_params=pltpu.CompilerParams(
        needs_layout_passes=False, use_tc_tiling_on_sc=False
    ),
)(_sc_hist_body)


def _tc_head_body(hist_ref, table_ref, wp_ref, bp_ref, wc_ref, bc_ref, out_ref):
    h = hist_ref[...]
    counts = jnp.sum(h[:, :N_VAL], axis=1, keepdims=True)
    sums = jnp.dot(h, table_ref[...], preferred_element_type=jnp.float32,
                   precision=lax.Precision.HIGHEST)
    pooled = sums / jnp.maximum(counts, 1.0)
    hidden = jnp.dot(pooled, wp_ref[...], preferred_element_type=jnp.float32,
                     precision=lax.Precision.HIGHEST) + bp_ref[...]
    hidden = jnp.maximum(hidden, 0.0)
    logits = jnp.dot(hidden, wc_ref[...], preferred_element_type=jnp.float32,
                     precision=lax.Precision.HIGHEST) + bc_ref[...]
    out_ref[...] = logits


_tc_head = pl.pallas_call(
    _tc_head_body,
    out_shape=jax.ShapeDtypeStruct((N_GRAPH, 128), jnp.float32),
)


def kernel(x, batch, shape_emb, color_emb, pos_emb, W_proj, b_proj, W_cls, b_cls):
    zeros16 = jnp.zeros((16, N_FEAT), jnp.float32)
    s = x[:, 0]
    c = x[:, 1]
    p = x[:, 2]
    hist = _sc_hist(s, c, p, batch, zeros16)
    table = jnp.concatenate([shape_emb, color_emb, pos_emb[:N_VAL]], axis=0)
    wc_pad = jnp.pad(W_cls, ((0, 0), (0, 128 - N_CLASS)))
    bc_pad = jnp.pad(b_cls, (0, 128 - N_CLASS)).reshape(1, 128)
    logits = _tc_head(hist, table, W_proj, b_proj.reshape(1, HID_DIM), wc_pad, bc_pad)
    return logits[:, :N_CLASS]


# speedup vs baseline: 1.0187x; 1.0187x over previous
import functools

import jax
import jax.numpy as jnp
from jax import lax
from jax.experimental import pallas as pl
from jax.experimental.pallas import tpu as pltpu
from jax.experimental.pallas import tpu_sc as plsc

N_TOK = 1048576
N_GRAPH = 1024
N_VAL = 64
N_FEAT = 192
EMB_DIM = 64
HID_DIM = 256
N_CLASS = 10

N_CORES = 2
N_SUBCORES = 16
GH = N_GRAPH // N_CORES
HIST_ROWS = GH + 16
TOK_PER_TILE = N_TOK // N_SUBCORES
CHUNK = 8192
N_CHUNK = TOK_PER_TILE // CHUNK
GROUPS = CHUNK // 16
NBUF = 2


def _sc_hist_body(key_hbm, zeros_hbm, out_hbm, k_v, hist_v, shared,
                  sem_z, sem_0, sem_1):
    cid = lax.axis_index("c")
    sid = lax.axis_index("s")
    t0 = sid * TOK_PER_TILE
    gbase = cid * GH

    rows_per_tile = GH // N_SUBCORES
    srow = pl.multiple_of(sid * rows_per_tile, 16)
    zdescs = []
    for r in range(HIST_ROWS // 16):
        zdescs.append(pltpu.async_copy(
            zeros_hbm, hist_v.at[pl.ds(r * 16, 16)], sem_z))
    zdescs.append(pltpu.async_copy(zeros_hbm, shared.at[pl.ds(srow, 16)], sem_z))
    zdescs.append(pltpu.async_copy(zeros_hbm, shared.at[pl.ds(srow + 16, 16)], sem_z))
    for d in zdescs:
        d.wait()
    plsc.subcore_barrier()

    viota = lax.iota(jnp.int32, 16)
    ones = jnp.full((16,), 1.0, jnp.float32)
    sems = (sem_0, sem_1)

    def issue(k):
        slot = k % NBUF
        off = pl.multiple_of(t0 + k * CHUNK, CHUNK)
        return pltpu.async_copy(
            key_hbm.at[pl.ds(off, CHUNK)], k_v.at[slot], sems[slot])

    descs = [None] * NBUF
    for k in range(min(NBUF - 1, N_CHUNK)):
        descs[k % NBUF] = issue(k)
    gfirst = jnp.int32(0)
    glast = jnp.int32(0)
    for k in range(N_CHUNK):
        slot = k % NBUF
        if k + NBUF - 1 < N_CHUNK:
            descs[(k + NBUF - 1) % NBUF] = issue(k + NBUF - 1)
        descs[slot].wait()
        kk = k_v.at[slot]

        def grp_body(i, c, kk=kk):
            j = pl.multiple_of(i * 16, 16)
            vk = kk[pl.ds(j, 16)]
            vg = lax.shift_right_logical(vk, 18)
            gl = vg - gbase
            msk = (gl >= 0) & (gl < GH)
            glc = jnp.minimum(jnp.maximum(gl, 0), GH - 1)
            sval = vk & 63
            cval = lax.shift_right_logical(vk, 6) & 63
            pval = lax.shift_right_logical(vk, 12) & 63
            plsc.addupdate_scatter(hist_v, [glc, sval], ones, mask=msk)
            plsc.addupdate_scatter(hist_v, [glc, cval + N_VAL], ones, mask=msk)
            plsc.addupdate_scatter(hist_v, [glc, pval + 2 * N_VAL], ones, mask=msk)
            return c
        lax.fori_loop(0, GROUPS, grp_body, 0)

        if k == 0:
            gfirst = lax.shift_right_logical(jnp.min(kk[pl.ds(0, 16)]), 18)
        if k == N_CHUNK - 1:
            glast = lax.shift_right_logical(jnp.max(kk[pl.ds(CHUNK - 16, 16)]), 18)

    lo = (jnp.clip(gfirst - gbase, 0, GH) // 16) * 16
    hi = jnp.clip(glast - gbase + 1, 0, GH)
    nwin = (hi - lo + 15) // 16

    def flush_body(t, c):
        r = pl.multiple_of(lo + t * 16, 16)
        rows = jnp.minimum(r + viota, GH - 1)
        pltpu.sync_copy(hist_v.at[pl.ds(r, 16)], shared.at[rows], add=True)
        return c
    lax.fori_loop(0, nwin, flush_body, 0)
    plsc.subcore_barrier()

    def out_body(t, c):
        r = pl.multiple_of(sid * rows_per_tile + t * 16, 16)
        pltpu.sync_copy(shared.at[pl.ds(r, 16)],
                        out_hbm.at[pl.ds(pl.multiple_of(gbase + r, 16), 16)])
        return c
    lax.fori_loop(0, rows_per_tile // 16, out_body, 0)


_sc_hist = functools.partial(
    pl.kernel,
    out_type=jax.ShapeDtypeStruct((N_GRAPH, N_FEAT), jnp.float32),
    mesh=plsc.VectorSubcoreMesh(
        core_axis_name="c", subcore_axis_name="s",
        num_cores=N_CORES, num_subcores=N_SUBCORES,
    ),
    scratch_types=[
        pltpu.VMEM((NBUF, CHUNK), jnp.int32),
        pltpu.VMEM((HIST_ROWS, N_FEAT), jnp.float32),
        pltpu.VMEM_SHARED((GH, N_FEAT), jnp.float32),
        pltpu.SemaphoreType.DMA,
        pltpu.SemaphoreType.DMA,
        pltpu.SemaphoreType.DMA,
    ],
    compiler_params=pltpu.CompilerParams(
        needs_layout_passes=False, use_tc_tiling_on_sc=False
    ),
)(_sc_hist_body)


def _tc_head_body(hist_ref, table_ref, wp_ref, bp_ref, wc_ref, bc_ref, out_ref):
    h = hist_ref[...]
    counts = jnp.sum(h[:, :N_VAL], axis=1, keepdims=True)
    sums = jnp.dot(h, table_ref[...], preferred_element_type=jnp.float32,
                   precision=lax.Precision.HIGHEST)
    pooled = sums / jnp.maximum(counts, 1.0)
    hidden = jnp.dot(pooled, wp_ref[...], preferred_element_type=jnp.float32,
                     precision=lax.Precision.HIGHEST) + bp_ref[...]
    hidden = jnp.maximum(hidden, 0.0)
    logits = jnp.dot(hidden, wc_ref[...], preferred_element_type=jnp.float32,
                     precision=lax.Precision.HIGHEST) + bc_ref[...]
    out_ref[...] = logits


_tc_head = pl.pallas_call(
    _tc_head_body,
    out_shape=jax.ShapeDtypeStruct((N_GRAPH, 128), jnp.float32),
)


def kernel(x, batch, shape_emb, color_emb, pos_emb, W_proj, b_proj, W_cls, b_cls):
    zeros16 = jnp.zeros((16, N_FEAT), jnp.float32)
    key = (
        jnp.left_shift(batch, 18)
        | jnp.left_shift(x[:, 2], 12)
        | jnp.left_shift(x[:, 1], 6)
        | x[:, 0]
    )
    hist = _sc_hist(key, zeros16)
    table = jnp.concatenate([shape_emb, color_emb, pos_emb[:N_VAL]], axis=0)
    wc_pad = jnp.pad(W_cls, ((0, 0), (0, 128 - N_CLASS)))
    bc_pad = jnp.pad(b_cls, (0, 128 - N_CLASS)).reshape(1, 128)
    logits = _tc_head(hist, table, W_proj, b_proj.reshape(1, HID_DIM), wc_pad, bc_pad)
    return logits[:, :N_CLASS]


# speedup vs baseline: 1.2515x; 1.2285x over previous
import functools

import jax
import jax.numpy as jnp
from jax import lax
from jax.experimental import pallas as pl
from jax.experimental.pallas import tpu as pltpu
from jax.experimental.pallas import tpu_sc as plsc

N_TOK = 1048576
N_GRAPH = 1024
N_VAL = 64
N_FEAT = 192
EMB_DIM = 64
HID_DIM = 256
N_CLASS = 10

N_CORES = 2
N_SUBCORES = 16
GH = N_GRAPH // N_CORES
HIST_ROWS = GH + 16
TOK_PER_TILE = N_TOK // N_SUBCORES
CHUNK = 8192
N_CHUNK = TOK_PER_TILE // CHUNK
GROUPS = CHUNK // 16
NBUF = 2


def _sc_hist_body(key_hbm, zeros_hbm, out_hbm, k_v, hist_v, shared,
                  sem_z, sem_0, sem_1):
    cid = lax.axis_index("c")
    sid = lax.axis_index("s")
    t0 = sid * TOK_PER_TILE
    gbase = cid * GH

    rows_per_tile = GH // N_SUBCORES
    srow = pl.multiple_of(sid * rows_per_tile, 16)
    d1 = pltpu.async_copy(zeros_hbm, hist_v, sem_z)
    d2 = pltpu.async_copy(zeros_hbm.at[pl.ds(0, rows_per_tile)],
                          shared.at[pl.ds(srow, rows_per_tile)], sem_z)
    d1.wait()
    d2.wait()
    plsc.subcore_barrier()

    viota = lax.iota(jnp.int32, 16)
    ones = jnp.full((16,), 1.0, jnp.float32)
    sems = (sem_0, sem_1)

    def issue(k):
        slot = k % NBUF
        off = pl.multiple_of(t0 + k * CHUNK, CHUNK)
        return pltpu.async_copy(
            key_hbm.at[pl.ds(off, CHUNK)], k_v.at[slot], sems[slot])

    descs = [None] * NBUF
    for k in range(min(NBUF - 1, N_CHUNK)):
        descs[k % NBUF] = issue(k)
    gfirst = jnp.int32(0)
    glast = jnp.int32(0)
    for k in range(N_CHUNK):
        slot = k % NBUF
        if k + NBUF - 1 < N_CHUNK:
            descs[(k + NBUF - 1) % NBUF] = issue(k + NBUF - 1)
        descs[slot].wait()
        kk = k_v.at[slot]

        def grp_body(i, c, kk=kk):
            j = pl.multiple_of(i * 16, 16)
            vk = kk[pl.ds(j, 16)]
            vg = lax.shift_right_logical(vk, 18)
            gl = vg - gbase
            msk = (gl >= 0) & (gl < GH)
            glc = jnp.minimum(jnp.maximum(gl, 0), GH - 1)
            sval = vk & 63
            cval = lax.shift_right_logical(vk, 6) & 63
            pval = lax.shift_right_logical(vk, 12) & 63
            plsc.addupdate_scatter(hist_v, [glc, sval], ones, mask=msk)
            plsc.addupdate_scatter(hist_v, [glc, cval + N_VAL], ones, mask=msk)
            plsc.addupdate_scatter(hist_v, [glc, pval + 2 * N_VAL], ones, mask=msk)
            return c
        lax.fori_loop(0, GROUPS, grp_body, 0)

        if k == 0:
            gfirst = lax.shift_right_logical(jnp.min(kk[pl.ds(0, 16)]), 18)
        if k == N_CHUNK - 1:
            glast = lax.shift_right_logical(jnp.max(kk[pl.ds(CHUNK - 16, 16)]), 18)

    lo = (jnp.clip(gfirst - gbase, 0, GH) // 16) * 16
    hi = jnp.clip(glast - gbase + 1, 0, GH)
    nwin = (hi - lo + 15) // 16

    def flush_body(t, c):
        r = pl.multiple_of(lo + t * 16, 16)
        rows = jnp.minimum(r + viota, GH - 1)
        pltpu.sync_copy(hist_v.at[pl.ds(r, 16)], shared.at[rows], add=True)
        return c
    lax.fori_loop(0, nwin, flush_body, 0)
    plsc.subcore_barrier()

    def out_body(t, c):
        r = pl.multiple_of(sid * rows_per_tile + t * 16, 16)
        pltpu.sync_copy(shared.at[pl.ds(r, 16)],
                        out_hbm.at[pl.ds(pl.multiple_of(gbase + r, 16), 16)])
        return c
    lax.fori_loop(0, rows_per_tile // 16, out_body, 0)


_sc_hist = functools.partial(
    pl.kernel,
    out_type=jax.ShapeDtypeStruct((N_GRAPH, N_FEAT), jnp.float32),
    mesh=plsc.VectorSubcoreMesh(
        core_axis_name="c", subcore_axis_name="s",
        num_cores=N_CORES, num_subcores=N_SUBCORES,
    ),
    scratch_types=[
        pltpu.VMEM((NBUF, CHUNK), jnp.int32),
        pltpu.VMEM((HIST_ROWS, N_FEAT), jnp.float32),
        pltpu.VMEM_SHARED((GH, N_FEAT), jnp.float32),
        pltpu.SemaphoreType.DMA,
        pltpu.SemaphoreType.DMA,
        pltpu.SemaphoreType.DMA,
    ],
    compiler_params=pltpu.CompilerParams(
        needs_layout_passes=False, use_tc_tiling_on_sc=False
    ),
)(_sc_hist_body)


def _tc_head_body(hist_ref, table_ref, wp_ref, bp_ref, wc_ref, bc_ref, out_ref):
    h = hist_ref[...]
    counts = jnp.sum(h[:, :N_VAL], axis=1, keepdims=True)
    sums = jnp.dot(h, table_ref[...], preferred_element_type=jnp.float32,
                   precision=lax.Precision.HIGHEST)
    pooled = sums / jnp.maximum(counts, 1.0)
    hidden = jnp.dot(pooled, wp_ref[...], preferred_element_type=jnp.float32,
                     precision=lax.Precision.HIGHEST) + bp_ref[...]
    hidden = jnp.maximum(hidden, 0.0)
    logits = jnp.dot(hidden, wc_ref[...], preferred_element_type=jnp.float32,
                     precision=lax.Precision.HIGHEST) + bc_ref[...]
    out_ref[...] = logits


_tc_head = pl.pallas_call(
    _tc_head_body,
    out_shape=jax.ShapeDtypeStruct((N_GRAPH, 128), jnp.float32),
)


def kernel(x, batch, shape_emb, color_emb, pos_emb, W_proj, b_proj, W_cls, b_cls):
    zeros_full = jnp.zeros((HIST_ROWS, N_FEAT), jnp.float32)
    key = (
        jnp.left_shift(batch, 18)
        | jnp.left_shift(x[:, 2], 12)
        | jnp.left_shift(x[:, 1], 6)
        | x[:, 0]
    )
    hist = _sc_hist(key, zeros_full)
    table = jnp.concatenate([shape_emb, color_emb, pos_emb[:N_VAL]], axis=0)
    wc_pad = jnp.pad(W_cls, ((0, 0), (0, 128 - N_CLASS)))
    bc_pad = jnp.pad(b_cls, (0, 128 - N_CLASS)).reshape(1, 128)
    logits = _tc_head(hist, table, W_proj, b_proj.reshape(1, HID_DIM), wc_pad, bc_pad)
    return logits[:, :N_CLASS]


# speedup vs baseline: 1.7481x; 1.3968x over previous
import functools

import jax
import jax.numpy as jnp
from jax import lax
from jax.experimental import pallas as pl
from jax.experimental.pallas import tpu as pltpu
from jax.experimental.pallas import tpu_sc as plsc

N_TOK = 1048576
N_GRAPH = 1024
N_VAL = 64
N_FEAT = 192
EMB_DIM = 64
HID_DIM = 256
N_CLASS = 10

N_CORES = 2
N_SUBCORES = 16
GH = N_GRAPH // N_CORES
HIST_ROWS = GH + 16
TOK_PER_TILE = N_TOK // N_SUBCORES
CHUNK = 8192
N_CHUNK = TOK_PER_TILE // CHUNK
GROUPS = CHUNK // 16
NBUF = 2


def _sc_hist_body(key_hbm, zeros_hbm, out_hbm, k_v, hist_v, shared,
                  sem_z, sem_0, sem_1):
    cid = lax.axis_index("c")
    sid = lax.axis_index("s")
    t0 = sid * TOK_PER_TILE
    gbase = cid * GH

    rows_per_tile = GH // N_SUBCORES
    srow = pl.multiple_of(sid * rows_per_tile, 16)
    d1 = pltpu.async_copy(zeros_hbm, hist_v, sem_z)
    d2 = pltpu.async_copy(zeros_hbm.at[pl.ds(0, rows_per_tile)],
                          shared.at[pl.ds(srow, rows_per_tile)], sem_z)
    d1.wait()
    d2.wait()
    plsc.subcore_barrier()

    viota = lax.iota(jnp.int32, 16)
    ones = jnp.full((16,), 1.0, jnp.float32)
    sems = (sem_0, sem_1)

    def issue(k):
        slot = k % NBUF
        off = pl.multiple_of(t0 + k * CHUNK, CHUNK)
        return pltpu.async_copy(
            key_hbm.at[pl.ds(off, CHUNK)], k_v.at[slot], sems[slot])

    descs = [None] * NBUF
    for k in range(min(NBUF - 1, N_CHUNK)):
        descs[k % NBUF] = issue(k)
    gfirst = jnp.int32(0)
    glast = jnp.int32(0)
    for k in range(N_CHUNK):
        slot = k % NBUF
        if k + NBUF - 1 < N_CHUNK:
            descs[(k + NBUF - 1) % NBUF] = issue(k + NBUF - 1)
        descs[slot].wait()
        kk = k_v.at[slot]

        @plsc.parallel_loop(0, CHUNK, 16, unroll=4)
        def grp_body(j, kk=kk):
            vk = kk[pl.ds(pl.multiple_of(j, 16), 16)]
            vg = lax.shift_right_logical(vk, 18)
            gl = vg - gbase
            msk = (gl >= 0) & (gl < GH)
            glc = jnp.minimum(jnp.maximum(gl, 0), GH - 1)
            sval = vk & 63
            cval = lax.shift_right_logical(vk, 6) & 63
            pval = lax.shift_right_logical(vk, 12) & 63
            plsc.addupdate_scatter(hist_v, [glc, sval], ones, mask=msk)
            plsc.addupdate_scatter(hist_v, [glc, cval + N_VAL], ones, mask=msk)
            plsc.addupdate_scatter(hist_v, [glc, pval + 2 * N_VAL], ones, mask=msk)

        if k == 0:
            gfirst = lax.shift_right_logical(jnp.min(kk[pl.ds(0, 16)]), 18)
        if k == N_CHUNK - 1:
            glast = lax.shift_right_logical(jnp.max(kk[pl.ds(CHUNK - 16, 16)]), 18)

    lo = (jnp.clip(gfirst - gbase, 0, GH) // 16) * 16
    hi = jnp.clip(glast - gbase + 1, 0, GH)
    nwin = (hi - lo + 15) // 16

    def flush_body(t, c):
        r = pl.multiple_of(lo + t * 16, 16)
        rows = jnp.minimum(r + viota, GH - 1)
        pltpu.sync_copy(hist_v.at[pl.ds(r, 16)], shared.at[rows], add=True)
        return c
    lax.fori_loop(0, nwin, flush_body, 0)
    plsc.subcore_barrier()

    def out_body(t, c):
        r = pl.multiple_of(sid * rows_per_tile + t * 16, 16)
        pltpu.sync_copy(shared.at[pl.ds(r, 16)],
                        out_hbm.at[pl.ds(pl.multiple_of(gbase + r, 16), 16)])
        return c
    lax.fori_loop(0, rows_per_tile // 16, out_body, 0)


_sc_hist = functools.partial(
    pl.kernel,
    out_type=jax.ShapeDtypeStruct((N_GRAPH, N_FEAT), jnp.float32),
    mesh=plsc.VectorSubcoreMesh(
        core_axis_name="c", subcore_axis_name="s",
        num_cores=N_CORES, num_subcores=N_SUBCORES,
    ),
    scratch_types=[
        pltpu.VMEM((NBUF, CHUNK), jnp.int32),
        pltpu.VMEM((HIST_ROWS, N_FEAT), jnp.float32),
        pltpu.VMEM_SHARED((GH, N_FEAT), jnp.float32),
        pltpu.SemaphoreType.DMA,
        pltpu.SemaphoreType.DMA,
        pltpu.SemaphoreType.DMA,
    ],
    compiler_params=pltpu.CompilerParams(
        needs_layout_passes=False, use_tc_tiling_on_sc=False
    ),
)(_sc_hist_body)


def _tc_head_body(hist_ref, table_ref, wp_ref, bp_ref, wc_ref, bc_ref, out_ref):
    h = hist_ref[...]
    counts = jnp.sum(h[:, :N_VAL], axis=1, keepdims=True)
    sums = jnp.dot(h, table_ref[...], preferred_element_type=jnp.float32,
                   precision=lax.Precision.HIGHEST)
    pooled = sums / jnp.maximum(counts, 1.0)
    hidden = jnp.dot(pooled, wp_ref[...], preferred_element_type=jnp.float32,
                     precision=lax.Precision.HIGHEST) + bp_ref[...]
    hidden = jnp.maximum(hidden, 0.0)
    logits = jnp.dot(hidden, wc_ref[...], preferred_element_type=jnp.float32,
                     precision=lax.Precision.HIGHEST) + bc_ref[...]
    out_ref[...] = logits


_tc_head = pl.pallas_call(
    _tc_head_body,
    out_shape=jax.ShapeDtypeStruct((N_GRAPH, 128), jnp.float32),
)


def kernel(x, batch, shape_emb, color_emb, pos_emb, W_proj, b_proj, W_cls, b_cls):
    zeros_full = jnp.zeros((HIST_ROWS, N_FEAT), jnp.float32)
    key = (
        jnp.left_shift(batch, 18)
        | jnp.left_shift(x[:, 2], 12)
        | jnp.left_shift(x[:, 1], 6)
        | x[:, 0]
    )
    hist = _sc_hist(key, zeros_full)
    table = jnp.concatenate([shape_emb, color_emb, pos_emb[:N_VAL]], axis=0)
    wc_pad = jnp.pad(W_cls, ((0, 0), (0, 128 - N_CLASS)))
    bc_pad = jnp.pad(b_cls, (0, 128 - N_CLASS)).reshape(1, 128)
    logits = _tc_head(hist, table, W_proj, b_proj.reshape(1, HID_DIM), wc_pad, bc_pad)
    return logits[:, :N_CLASS]


# speedup vs baseline: 1.7582x; 1.0058x over previous
import functools

import jax
import jax.numpy as jnp
from jax import lax
from jax.experimental import pallas as pl
from jax.experimental.pallas import tpu as pltpu
from jax.experimental.pallas import tpu_sc as plsc

N_TOK = 1048576
N_GRAPH = 1024
N_VAL = 64
N_FEAT = 192
EMB_DIM = 64
HID_DIM = 256
N_CLASS = 10

N_CORES = 2
N_SUBCORES = 16
GH = N_GRAPH // N_CORES
HIST_ROWS = GH + 16
TOK_PER_TILE = N_TOK // N_SUBCORES
CHUNK = 8192
N_CHUNK = TOK_PER_TILE // CHUNK
GROUPS = CHUNK // 16
NBUF = 2


def _sc_hist_body(key_hbm, zeros_hbm, out_hbm, k_v, hist_v, shared,
                  sem_z, sem_0, sem_1):
    cid = lax.axis_index("c")
    sid = lax.axis_index("s")
    t0 = sid * TOK_PER_TILE
    gbase = cid * GH

    rows_per_tile = GH // N_SUBCORES
    srow = pl.multiple_of(sid * rows_per_tile, 16)
    d1 = pltpu.async_copy(zeros_hbm, hist_v, sem_z)
    d2 = pltpu.async_copy(zeros_hbm.at[pl.ds(0, rows_per_tile)],
                          shared.at[pl.ds(srow, rows_per_tile)], sem_z)
    d1.wait()
    d2.wait()
    plsc.subcore_barrier()

    viota = lax.iota(jnp.int32, 16)
    ones = jnp.full((16,), 1.0, jnp.float32)
    sems = (sem_0, sem_1)

    def issue(k):
        slot = k % NBUF
        off = pl.multiple_of(t0 + k * CHUNK, CHUNK)
        return pltpu.async_copy(
            key_hbm.at[pl.ds(off, CHUNK)], k_v.at[slot], sems[slot])

    descs = [None] * NBUF
    for k in range(min(NBUF - 1, N_CHUNK)):
        descs[k % NBUF] = issue(k)
    gfirst = jnp.int32(0)
    glast = jnp.int32(0)
    for k in range(N_CHUNK):
        slot = k % NBUF
        if k + NBUF - 1 < N_CHUNK:
            descs[(k + NBUF - 1) % NBUF] = issue(k + NBUF - 1)
        descs[slot].wait()
        kk = k_v.at[slot]

        @plsc.parallel_loop(0, CHUNK, 16, unroll=8)
        def grp_body(j, kk=kk):
            vk = kk[pl.ds(pl.multiple_of(j, 16), 16)]
            vg = lax.shift_right_logical(vk, 18)
            gl = vg - gbase
            msk = (gl >= 0) & (gl < GH)
            glc = jnp.minimum(jnp.maximum(gl, 0), GH - 1)
            sval = vk & 63
            cval = lax.shift_right_logical(vk, 6) & 63
            pval = lax.shift_right_logical(vk, 12) & 63
            plsc.addupdate_scatter(hist_v, [glc, sval], ones, mask=msk)
            plsc.addupdate_scatter(hist_v, [glc, cval + N_VAL], ones, mask=msk)
            plsc.addupdate_scatter(hist_v, [glc, pval + 2 * N_VAL], ones, mask=msk)

        if k == 0:
            gfirst = lax.shift_right_logical(jnp.min(kk[pl.ds(0, 16)]), 18)
        if k == N_CHUNK - 1:
            glast = lax.shift_right_logical(jnp.max(kk[pl.ds(CHUNK - 16, 16)]), 18)

    lo = (jnp.clip(gfirst - gbase, 0, GH) // 16) * 16
    hi = jnp.clip(glast - gbase + 1, 0, GH)
    nwin = (hi - lo + 15) // 16

    def flush_body(t, c):
        r = pl.multiple_of(lo + t * 16, 16)
        rows = jnp.minimum(r + viota, GH - 1)
        pltpu.sync_copy(hist_v.at[pl.ds(r, 16)], shared.at[rows], add=True)
        return c
    lax.fori_loop(0, nwin, flush_body, 0)
    plsc.subcore_barrier()

    def out_body(t, c):
        r = pl.multiple_of(sid * rows_per_tile + t * 16, 16)
        pltpu.sync_copy(shared.at[pl.ds(r, 16)],
                        out_hbm.at[pl.ds(pl.multiple_of(gbase + r, 16), 16)])
        return c
    lax.fori_loop(0, rows_per_tile // 16, out_body, 0)


_sc_hist = functools.partial(
    pl.kernel,
    out_type=jax.ShapeDtypeStruct((N_GRAPH, N_FEAT), jnp.float32),
    mesh=plsc.VectorSubcoreMesh(
        core_axis_name="c", subcore_axis_name="s",
        num_cores=N_CORES, num_subcores=N_SUBCORES,
    ),
    scratch_types=[
        pltpu.VMEM((NBUF, CHUNK), jnp.int32),
        pltpu.VMEM((HIST_ROWS, N_FEAT), jnp.float32),
        pltpu.VMEM_SHARED((GH, N_FEAT), jnp.float32),
        pltpu.SemaphoreType.DMA,
        pltpu.SemaphoreType.DMA,
        pltpu.SemaphoreType.DMA,
    ],
    compiler_params=pltpu.CompilerParams(
        needs_layout_passes=False, use_tc_tiling_on_sc=False
    ),
)(_sc_hist_body)


def _tc_head_body(hist_ref, table_ref, wp_ref, bp_ref, wc_ref, bc_ref, out_ref):
    h = hist_ref[...]
    counts = jnp.sum(h[:, :N_VAL], axis=1, keepdims=True)
    sums = jnp.dot(h, table_ref[...], preferred_element_type=jnp.float32,
                   precision=lax.Precision.HIGHEST)
    pooled = sums / jnp.maximum(counts, 1.0)
    hidden = jnp.dot(pooled, wp_ref[...], preferred_element_type=jnp.float32,
                     precision=lax.Precision.HIGHEST) + bp_ref[...]
    hidden = jnp.maximum(hidden, 0.0)
    logits = jnp.dot(hidden, wc_ref[...], preferred_element_type=jnp.float32,
                     precision=lax.Precision.HIGHEST) + bc_ref[...]
    out_ref[...] = logits


_tc_head = pl.pallas_call(
    _tc_head_body,
    out_shape=jax.ShapeDtypeStruct((N_GRAPH, 128), jnp.float32),
)


def kernel(x, batch, shape_emb, color_emb, pos_emb, W_proj, b_proj, W_cls, b_cls):
    zeros_full = jnp.zeros((HIST_ROWS, N_FEAT), jnp.float32)
    key = (
        jnp.left_shift(batch, 18)
        | jnp.left_shift(x[:, 2], 12)
        | jnp.left_shift(x[:, 1], 6)
        | x[:, 0]
    )
    hist = _sc_hist(key, zeros_full)
    table = jnp.concatenate([shape_emb, color_emb, pos_emb[:N_VAL]], axis=0)
    wc_pad = jnp.pad(W_cls, ((0, 0), (0, 128 - N_CLASS)))
    bc_pad = jnp.pad(b_cls, (0, 128 - N_CLASS)).reshape(1, 128)
    logits = _tc_head(hist, table, W_proj, b_proj.reshape(1, HID_DIM), wc_pad, bc_pad)
    return logits[:, :N_CLASS]
